# baseline (speedup 1.0000x reference)
import functools

import numpy as np

import jax
import jax.numpy as jnp
from jax import lax
from jax.experimental import pallas as pl
from jax.experimental.pallas import tpu as pltpu
from jax.experimental.pallas import tpu_sc as plsc

_info = plsc.get_sparse_core_info()
_NC, _NS, _L = _info.num_cores, _info.num_subcores, _info.num_lanes
_NW = _NC * _NS
_SW = 16
_BW = _NW // _SW
_G = 32


def _combine_body(tok_ref, trk_ref, out_ref):
    t = tok_ref[...]
    out_ref[0] = (t + trk_ref[0:1, :]).astype(jnp.bfloat16)
    out_ref[1] = (t + trk_ref[1:2, :]).astype(jnp.bfloat16)


def _make_sc_call(B, S, V, D):
    SCH = S // _SW
    BCH = B // _BW
    H = SCH // _G
    DH = D // 2
    mesh = plsc.VectorSubcoreMesh(core_axis_name="c", subcore_axis_name="s")

    def _sc_body(tok_hbm, trk_hbm, ctab_hbm, pee_hbm, peo_hbm, out_hbm,
                 idx_v, trk_v, pee_v, peo_v, buf0, buf1, obuf0, obuf1,
                 g_sem, w_sem):
        wid = lax.axis_index("s") * _NC + lax.axis_index("c")
        s0 = pl.multiple_of((wid % _SW) * SCH, SCH)
        b0 = pl.multiple_of((wid // _SW) * BCH, BCH)

        pltpu.sync_copy(tok_hbm.at[pl.ds(b0, BCH), pl.ds(s0, SCH)], idx_v)
        pltpu.sync_copy(trk_hbm.at[pl.ds(b0, BCH), pl.ds(s0, SCH)], trk_v)

        def idx_body(i, carry):
            for c in range(SCH // _L):
                sl = pl.ds(c * _L, _L)
                idx_v[i, sl] = trk_v[i, sl] * V + idx_v[i, sl]
            return carry

        lax.fori_loop(0, BCH, idx_body, 0)

        lane2 = jax.lax.iota(jnp.int32, _L) * 2

        def _wait_gather(buf):
            pltpu.make_async_copy(ctab_hbm.at[pl.ds(0, _G)], buf, g_sem).wait()

        def _wait_write(obuf):
            pltpu.make_async_copy(
                obuf, out_hbm.at[pl.ds(0, _G * D)], w_sem
            ).wait()

        def _start_gather(b, buf, h):
            pltpu.async_copy(
                ctab_hbm.at[idx_v.at[b, pl.ds(h * _G, _G)]], buf, g_sem
            )

        def _compute_and_write(b, buf, obuf, h):
            def row_body(i, c2):
                base = i * D
                for g in range(D // (2 * _L)):
                    x = buf[i, pl.ds(g * _L, _L)]
                    y = plsc.bitcast(x, jnp.bfloat16)
                    e, o = plsc.unpack(
                        y, format=plsc.PackFormat.INTERLEAVED
                    )
                    sl = pl.ds(g * _L, _L)
                    e = e + pee_v[i, sl]
                    o = o + peo_v[i, sl]
                    ie = lane2 + (base + 2 * _L * g)
                    plsc.store_scatter(obuf, [ie], e)
                    plsc.store_scatter(obuf, [ie + 1], o)
                return c2

            lax.fori_loop(0, _G, row_body, 0)
            r0 = ((b0 + b) * S + s0 + h * _G) * D
            pltpu.async_copy(obuf, out_hbm.at[pl.ds(r0, _G * D)], w_sem)

        for h in range(H):
            _start_gather(0, buf0, h)
            pltpu.sync_copy(pee_hbm.at[pl.ds(s0 + h * _G, _G)], pee_v)
            pltpu.sync_copy(peo_hbm.at[pl.ds(s0 + h * _G, _G)], peo_v)

            def pair_body(k, carry):
                b = 2 * k
                _wait_gather(buf0)
                _start_gather(b + 1, buf1, h)

                @pl.when(k >= 1)
                def _():
                    _wait_write(obuf0)

                _compute_and_write(b, buf0, obuf0, h)

                _wait_gather(buf1)

                @pl.when(k < BCH // 2 - 1)
                def _():
                    _start_gather(b + 2, buf0, h)

                @pl.when(k >= 1)
                def _():
                    _wait_write(obuf1)

                _compute_and_write(b + 1, buf1, obuf1, h)
                return carry

            lax.fori_loop(0, BCH // 2, pair_body, 0)
            _wait_write(obuf0)
            _wait_write(obuf1)

    return pl.kernel(
        _sc_body,
        out_type=jax.ShapeDtypeStruct((B * S * D,), jnp.float32),
        mesh=mesh,
        compiler_params=pltpu.CompilerParams(needs_layout_passes=False),
        scratch_types=[
            pltpu.VMEM((BCH, SCH), jnp.int32),
            pltpu.VMEM((BCH, SCH), jnp.int32),
            pltpu.VMEM((_G, DH), jnp.float32),
            pltpu.VMEM((_G, DH), jnp.float32),
            pltpu.VMEM((_G, D // 2), jnp.int32),
            pltpu.VMEM((_G, D // 2), jnp.int32),
            pltpu.VMEM((_G * D,), jnp.float32),
            pltpu.VMEM((_G * D,), jnp.float32),
            pltpu.SemaphoreType.DMA,
            pltpu.SemaphoreType.DMA,
        ],
    )


def kernel(token_ids, track_ids, token_table, track_table, pe):
    B, S = token_ids.shape
    V, D = token_table.shape
    T = track_table.shape[0]

    tok = token_ids.astype(jnp.int32)
    trk = track_ids.astype(jnp.int32)

    ctab = pl.pallas_call(
        _combine_body,
        out_shape=jax.ShapeDtypeStruct((T, V, D), jnp.bfloat16),
    )(token_table, track_table)
    ctab = lax.bitcast_convert_type(
        ctab.reshape(T * V, D // 2, 2), jnp.int32
    )

    pe2d = pe.reshape(pe.shape[1], D)[:S]
    pe_e = pe2d[:, 0::2]
    pe_o = pe2d[:, 1::2]

    out = _make_sc_call(B, S, V, D)(tok, trk, ctab, pe_e, pe_o)
    return out.reshape(B, S, D)
